# baseline (speedup 1.0000x reference)
import functools

import jax
import jax.numpy as jnp
from jax import lax
from jax.experimental import pallas as pl
from jax.experimental.pallas import tpu as pltpu
from jax.experimental.pallas import tpu_sc as plsc

NUM_CORES = 2
NUM_SUBCORES = 16
NUM_WORKERS = NUM_CORES * NUM_SUBCORES
PAD = 128
LANES = 16


def _make_fn(seq, batch, dim):
    b_per_w = batch // NUM_WORKERS
    assert batch % NUM_WORKERS == 0 and seq % 2 == 0 and b_per_w % LANES == 0
    mesh = plsc.VectorSubcoreMesh(core_axis_name="c", subcore_axis_name="s")

    @functools.partial(
        pl.kernel,
        mesh=mesh,
        out_type=jax.ShapeDtypeStruct((seq, dim, batch), jnp.float32),
        scratch_types=[
            pltpu.VMEM((seq, b_per_w), jnp.int32),
            [pltpu.VMEM((b_per_w, PAD), jnp.float32) for _ in range(2)],
            [pltpu.VMEM((dim, b_per_w), jnp.float32) for _ in range(2)],
            [pltpu.SemaphoreType.DMA for _ in range(2)],
            [pltpu.SemaphoreType.DMA for _ in range(2)],
        ],
        compiler_params=pltpu.CompilerParams(needs_layout_passes=False),
    )
    def fn(ids_hbm, table_hbm, out_hbm, idx_v, rows, trans, gsems, wsems):
        wid = lax.axis_index("s") * NUM_CORES + lax.axis_index("c")
        b0 = pl.multiple_of(wid * b_per_w, b_per_w)
        pltpu.sync_copy(ids_hbm.at[:, pl.ds(b0, b_per_w)], idx_v)
        iota = lax.iota(jnp.int32, LANES)

        def start_gather(s, slot):
            pltpu.async_copy(table_hbm.at[idx_v.at[s]], rows[slot], gsems[slot])

        def wait_gather(slot):
            pltpu.make_async_copy(
                table_hbm.at[idx_v.at[0]], rows[slot], gsems[slot]
            ).wait()

        def wait_write(slot):
            pltpu.make_async_copy(
                trans[slot], out_hbm.at[0, :, pl.ds(0, b_per_w)], wsems[slot]
            ).wait()

        start_gather(0, 0)

        def step(s, slot, other):
            wait_gather(slot)

            @pl.when(s + 1 < seq)
            def _():
                start_gather(s + 1, other)

            @pl.when(s >= 2)
            def _():
                wait_write(slot)

            def tgroup(k, carry):
                row_idx = k * LANES + iota
                for c in range(dim):
                    vals = plsc.load_gather(
                        rows[slot],
                        [row_idx, jnp.full((LANES,), c, jnp.int32)],
                    )
                    trans[slot][c, pl.ds(k * LANES, LANES)] = vals
                return carry

            lax.fori_loop(0, b_per_w // LANES, tgroup, 0)
            pltpu.async_copy(
                trans[slot], out_hbm.at[s, :, pl.ds(b0, b_per_w)], wsems[slot]
            )

        def outer(i, carry):
            step(2 * i, 0, 1)
            step(2 * i + 1, 1, 0)
            return carry

        lax.fori_loop(0, seq // 2, outer, 0)
        for slot in range(2):
            wait_write(slot)

    return fn


def kernel(token_ids, embedding_matrix):
    batch, seq = token_ids.shape
    num_rows, dim = embedding_matrix.shape
    ids_t = token_ids.T
    table128 = jnp.pad(embedding_matrix, ((0, 0), (0, PAD - dim)))
    fn = _make_fn(seq, batch, dim)
    out_t = fn(ids_t, table128)
    return jnp.transpose(out_t, (2, 0, 1))


# speedup vs baseline: 1.2581x; 1.2581x over previous
import functools

import jax
import jax.numpy as jnp
from jax import lax
from jax.experimental import pallas as pl
from jax.experimental.pallas import tpu as pltpu
from jax.experimental.pallas import tpu_sc as plsc

NUM_CORES = 2
NUM_SUBCORES = 16
NUM_WORKERS = NUM_CORES * NUM_SUBCORES
PAD = 128
LANES = 16


def _make_fn(seq, batch, dim):
    b_per_w = batch // NUM_WORKERS
    assert batch % NUM_WORKERS == 0 and seq % 2 == 0 and b_per_w % LANES == 0
    mesh = plsc.VectorSubcoreMesh(core_axis_name="c", subcore_axis_name="s")

    @functools.partial(
        pl.kernel,
        mesh=mesh,
        out_type=jax.ShapeDtypeStruct((seq, dim, batch), jnp.float32),
        scratch_types=[
            pltpu.VMEM((seq, b_per_w), jnp.int32),
            [pltpu.VMEM((b_per_w, PAD), jnp.float32) for _ in range(2)],
            [pltpu.VMEM((dim, b_per_w), jnp.float32) for _ in range(2)],
            [pltpu.SemaphoreType.DMA for _ in range(2)],
            [pltpu.SemaphoreType.DMA for _ in range(2)],
        ],
        compiler_params=pltpu.CompilerParams(needs_layout_passes=False),
    )
    def fn(ids_hbm, table_hbm, out_hbm, idx_v, rows, trans, gsems, wsems):
        wid = lax.axis_index("s") * NUM_CORES + lax.axis_index("c")
        b0 = pl.multiple_of(wid * b_per_w, b_per_w)
        pltpu.sync_copy(ids_hbm.at[:, pl.ds(b0, b_per_w)], idx_v)
        iota = lax.iota(jnp.int32, LANES)

        def start_gather(s, slot):
            pltpu.async_copy(table_hbm.at[idx_v.at[s]], rows[slot], gsems[slot])

        def wait_gather(slot):
            pltpu.make_async_copy(
                table_hbm.at[idx_v.at[0]], rows[slot], gsems[slot]
            ).wait()

        def wait_write(slot):
            pltpu.make_async_copy(
                trans[slot], out_hbm.at[0, :, pl.ds(0, b_per_w)], wsems[slot]
            ).wait()

        start_gather(0, 0)

        def step(s, slot, other):
            wait_gather(slot)

            @pl.when(s + 1 < seq)
            def _():
                start_gather(s + 1, other)

            @pl.when(s >= 2)
            def _():
                wait_write(slot)

            @plsc.parallel_loop(0, b_per_w // LANES, unroll=2)
            def tgroup(k):
                row_idx = k * LANES + iota
                for c in range(dim):
                    vals = plsc.load_gather(
                        rows[slot],
                        [row_idx, jnp.full((LANES,), c, jnp.int32)],
                    )
                    trans[slot][c, pl.ds(k * LANES, LANES)] = vals
            pltpu.async_copy(
                trans[slot], out_hbm.at[s, :, pl.ds(b0, b_per_w)], wsems[slot]
            )

        def outer(i, carry):
            step(2 * i, 0, 1)
            step(2 * i + 1, 1, 0)
            return carry

        lax.fori_loop(0, seq // 2, outer, 0)
        for slot in range(2):
            wait_write(slot)

    return fn


def kernel(token_ids, embedding_matrix):
    batch, seq = token_ids.shape
    num_rows, dim = embedding_matrix.shape
    ids_t = token_ids.T
    table128 = jnp.pad(embedding_matrix, ((0, 0), (0, PAD - dim)))
    fn = _make_fn(seq, batch, dim)
    out_t = fn(ids_t, table128)
    return jnp.transpose(out_t, (2, 0, 1))


# speedup vs baseline: 1.5616x; 1.2413x over previous
import functools

import jax
import jax.numpy as jnp
from jax import lax
from jax.experimental import pallas as pl
from jax.experimental.pallas import tpu as pltpu
from jax.experimental.pallas import tpu_sc as plsc

NUM_CORES = 2
NUM_SUBCORES = 16
NUM_WORKERS = NUM_CORES * NUM_SUBCORES
PAIR = 128
LANES = 16

CHUNK = 128
NBUF = 2


def _make_fn(total, dim):
    per_w = total // NUM_WORKERS
    assert total % NUM_WORKERS == 0 and per_w % (CHUNK * NBUF) == 0
    n_chunks = per_w // CHUNK
    mesh = plsc.VectorSubcoreMesh(core_axis_name="c", subcore_axis_name="s")

    @functools.partial(
        pl.kernel,
        mesh=mesh,
        out_type=jax.ShapeDtypeStruct((total, dim), jnp.float32),
        scratch_types=[
            pltpu.VMEM((per_w,), jnp.int32),
            [pltpu.VMEM((CHUNK,), jnp.int32) for _ in range(NBUF)],
            [pltpu.VMEM((CHUNK, PAIR), jnp.float32) for _ in range(NBUF)],
            [pltpu.VMEM((CHUNK, dim), jnp.float32) for _ in range(NBUF)],
            [pltpu.SemaphoreType.DMA for _ in range(NBUF)],
            [pltpu.SemaphoreType.DMA for _ in range(NBUF)],
        ],
        compiler_params=pltpu.CompilerParams(needs_layout_passes=False),
    )
    def fn(idx_hbm, table_hbm, out_hbm, idx_v, idxp, rows, comp, gsems, wsems):
        wid = lax.axis_index("s") * NUM_CORES + lax.axis_index("c")
        base = pl.multiple_of(wid * per_w, CHUNK)
        pltpu.sync_copy(idx_hbm.at[pl.ds(base, per_w)], idx_v)

        def start_gather(g, slot):
            off = pl.multiple_of(g * CHUNK, CHUNK)

            @plsc.parallel_loop(0, CHUNK // LANES, unroll=2)
            def _(k):
                ids = idx_v[pl.ds(off + k * LANES, LANES)]
                idxp[slot][pl.ds(k * LANES, LANES)] = ids >> 1

            pltpu.async_copy(table_hbm.at[idxp[slot]], rows[slot], gsems[slot])

        def wait_gather(slot):
            pltpu.make_async_copy(
                table_hbm.at[idxp[slot]], rows[slot], gsems[slot]
            ).wait()

        def wait_write(slot):
            pltpu.make_async_copy(
                comp[slot], out_hbm.at[pl.ds(0, CHUNK)], wsems[slot]
            ).wait()

        start_gather(0, 0)

        def step(g, slot, other):
            wait_gather(slot)

            @pl.when(g + 1 < n_chunks)
            def _():
                start_gather(g + 1, other)

            @pl.when(g >= NBUF)
            def _():
                wait_write(slot)

            off = pl.multiple_of(g * CHUNK, CHUNK)

            @plsc.parallel_loop(0, CHUNK // LANES, unroll=2)
            def _(k):
                tokv = idx_v[pl.ds(off + k * LANES, LANES)]
                hv = (tokv & 1) * dim
                for i in range(LANES):
                    h = hv[i]
                    t = k * LANES + i
                    for f in range(dim // LANES):
                        comp[slot][t, pl.ds(f * LANES, LANES)] = (
                            rows[slot][t, pl.ds(h + f * LANES, LANES)]
                        )

            pltpu.async_copy(
                comp[slot], out_hbm.at[pl.ds(base + off, CHUNK)], wsems[slot]
            )

        def outer(i, carry):
            step(NBUF * i, 0, 1)
            step(NBUF * i + 1, 1, 0)
            return carry

        lax.fori_loop(0, n_chunks // NBUF, outer, 0)
        for slot in range(NBUF):
            wait_write(slot)

    return fn


def kernel(token_ids, embedding_matrix):
    batch, seq = token_ids.shape
    num_rows, dim = embedding_matrix.shape
    flat_ids = token_ids.reshape(batch * seq)
    table_pairs = embedding_matrix.reshape(num_rows // 2, 2 * dim)
    fn = _make_fn(batch * seq, dim)
    out = fn(flat_ids, table_pairs)
    return out.reshape(batch, seq, dim)


# speedup vs baseline: 1.7397x; 1.1141x over previous
import functools

import jax
import jax.numpy as jnp
from jax import lax
from jax.experimental import pallas as pl
from jax.experimental.pallas import tpu as pltpu
from jax.experimental.pallas import tpu_sc as plsc

NUM_CORES = 2
NUM_SUBCORES = 16
NUM_WORKERS = NUM_CORES * NUM_SUBCORES

CHUNK = 256
NBUF = 2


def _make_gather(total, dim, dtype):
    assert total % (NUM_WORKERS * CHUNK * NBUF) == 0
    per_w = total // NUM_WORKERS
    n_chunks = per_w // CHUNK
    n_outer = n_chunks // NBUF
    mesh = plsc.VectorSubcoreMesh(core_axis_name="c", subcore_axis_name="s")

    @functools.partial(
        pl.kernel,
        mesh=mesh,
        out_type=jax.ShapeDtypeStruct((total, dim), dtype),
        scratch_types=[
            pltpu.VMEM((per_w,), jnp.int32),
            [pltpu.VMEM((CHUNK, dim), dtype) for _ in range(NBUF)],
            pltpu.SemaphoreType.DMA,
            [pltpu.SemaphoreType.DMA for _ in range(NBUF)],
        ],
    )
    def gather_kernel(idx_hbm, table_hbm, out_hbm, idx_v, rows, gsem, wsems):
        wid = lax.axis_index("s") * NUM_CORES + lax.axis_index("c")
        base = pl.multiple_of(wid * per_w, CHUNK)
        pltpu.sync_copy(idx_hbm.at[pl.ds(base, per_w)], idx_v)

        def do_chunk(g, b, first):
            off = pl.multiple_of(g * CHUNK, CHUNK)
            if not first:
                pltpu.make_async_copy(
                    rows[b], out_hbm.at[pl.ds(0, CHUNK)], wsems[b]
                ).wait()
            pltpu.async_copy(
                table_hbm.at[idx_v.at[pl.ds(off, CHUNK)]], rows[b], gsem
            ).wait()
            pltpu.async_copy(
                rows[b], out_hbm.at[pl.ds(base + off, CHUNK)], wsems[b]
            )

        for b in range(NBUF):
            do_chunk(b, b, first=True)

        def outer(i, carry):
            for b in range(NBUF):
                do_chunk(i * NBUF + b, b, first=False)
            return carry

        lax.fori_loop(1, n_outer, outer, 0)

        for b in range(NBUF):
            pltpu.make_async_copy(
                rows[b], out_hbm.at[pl.ds(0, CHUNK)], wsems[b]
            ).wait()

    return gather_kernel


def kernel(token_ids, embedding_matrix):
    batch, seq = token_ids.shape
    num_rows, dim = embedding_matrix.shape
    pad_dim = 128
    flat_ids = token_ids.reshape(batch * seq)
    table128 = jnp.pad(embedding_matrix, ((0, 0), (0, pad_dim - dim)))
    fn = _make_gather(batch * seq, pad_dim, embedding_matrix.dtype)
    out = fn(flat_ids, table128)
    return out[:, :dim].reshape(batch, seq, dim)
